# Initial kernel scaffold; baseline (speedup 1.0000x reference)
import jax
import jax.numpy as jnp
from jax.experimental import pallas as pl


def kernel(x, edge_index, rbf, W1, Wc1, bc1, Wc2, bc2, W2, b2, W3, b3):
    raise NotImplementedError("write your pallas kernel here")


# speedup vs baseline: 2.8216x; 2.8216x over previous
import functools

import jax
import jax.numpy as jnp
from jax import lax
from jax.experimental import pallas as pl
from jax.experimental.pallas import tpu as pltpu
from jax.experimental.pallas import tpu_sc as plsc

NC = 2
NS = 16
NW = NC * NS
K = 80


def _ssp(v):
    z = 0.5 * v
    return 2.0 * (jnp.maximum(z, 0.0) + jnp.log1p(jnp.exp(-jnp.abs(z))))


def _mm_t(a, b):
    return lax.dot_general(a, b, (((1,), (1,)), ((), ())),
                           preferred_element_type=jnp.float32)


def _nn_body(x_ref, w_ref, o_ref):
    o_ref[...] = _mm_t(x_ref[...], w_ref[...])


def _emlp_body(rbf_ref, wc1_ref, bc1_ref, wc2_ref, bc2_ref, h_ref):
    a = _ssp(_mm_t(rbf_ref[...], wc1_ref[...]) + bc1_ref[...])
    h_ref[...] = _mm_t(a, wc2_ref[...]) + bc2_ref[...]


def _out_body(p_ref, x_ref, w2_ref, b2_ref, w3_ref, b3_ref, o_ref):
    cf = p_ref[0] + p_ref[1]
    t = _ssp(_mm_t(cf, w2_ref[...]) + b2_ref[...])
    o_ref[...] = x_ref[...] + _mm_t(t, w3_ref[...]) + b3_ref[...]


@functools.cache
def _make_sc(N, D, E):
    EW = E // NW
    CH = EW // K
    RT = (N // NS) // 8 * 8
    TAIL = N - NS * RT
    mesh = plsc.VectorSubcoreMesh(core_axis_name="c", subcore_axis_name="s")

    @functools.partial(
        pl.kernel,
        out_type=jax.ShapeDtypeStruct((NC, N, D), jnp.float32),
        mesh=mesh,
        scratch_types=[
            pltpu.VMEM((1, K), jnp.int32),
            pltpu.VMEM((1, K), jnp.int32),
            pltpu.VMEM((K, D), jnp.float32),
            pltpu.VMEM((K, D), jnp.float32),
            pltpu.VMEM_SHARED((N, D), jnp.float32),
            pltpu.SemaphoreType.DMA,
            pltpu.SemaphoreType.DMA,
            pltpu.SemaphoreType.DMA,
            pltpu.SemaphoreType.DMA,
        ],
    )
    def sc_fn(nn_hbm, h_hbm, src_hbm, dst_hbm, zero_hbm, out_hbm,
              src_v, dst_v, gath_v, h_v, acc_sh, sem_s, sem_d, sem_g, sem_h):
        cid = lax.axis_index("c")
        sid = lax.axis_index("s")
        wid = cid * NS + sid

        pltpu.sync_copy(zero_hbm.at[pl.ds(sid * RT, RT)],
                        acc_sh.at[pl.ds(sid * RT, RT)])
        @pl.when(sid == NS - 1)
        def _():
            pltpu.sync_copy(zero_hbm.at[pl.ds(NS * RT, TAIL)],
                            acc_sh.at[pl.ds(NS * RT, TAIL)])
        plsc.subcore_barrier()

        base = wid * EW

        def chunk(c, carry):
            g = wid * CH + c
            scp = pltpu.async_copy(src_hbm.at[g], src_v, sem_s)
            dcp = pltpu.async_copy(dst_hbm.at[g], dst_v, sem_d)
            hcp = pltpu.async_copy(h_hbm.at[pl.ds(base + c * K, K)], h_v, sem_h)
            scp.wait()
            gcp = pltpu.async_copy(nn_hbm.at[src_v.at[0]], gath_v, sem_g)
            gcp.wait()
            hcp.wait()

            def row(r, carry2):
                for j in range(D // 16):
                    s = pl.ds(j * 16, 16)
                    gath_v[r, s] = gath_v[r, s] * h_v[r, s]
                return carry2

            lax.fori_loop(0, K, row, 0)
            dcp.wait()
            pltpu.sync_copy(gath_v, acc_sh.at[dst_v.at[0]], add=True)
            return carry

        lax.fori_loop(0, CH, chunk, 0)
        plsc.subcore_barrier()

        pltpu.sync_copy(acc_sh.at[pl.ds(sid * RT, RT)],
                        out_hbm.at[cid, pl.ds(sid * RT, RT)])
        @pl.when(sid == NS - 1)
        def _():
            pltpu.sync_copy(acc_sh.at[pl.ds(NS * RT, TAIL)],
                            out_hbm.at[cid, pl.ds(NS * RT, TAIL)])

    return sc_fn


def kernel(x, edge_index, rbf, W1, Wc1, bc1, Wc2, bc2, W2, b2, W3, b3):
    N, D = x.shape
    E, R = rbf.shape
    EW = E // NW
    CH = EW // K

    new_node = pl.pallas_call(
        _nn_body,
        out_shape=jax.ShapeDtypeStruct((N, D), jnp.float32),
    )(x, W1)

    BE = 4000
    h = pl.pallas_call(
        _emlp_body,
        grid=(E // BE,),
        in_specs=[
            pl.BlockSpec((BE, R), lambda i: (i, 0)),
            pl.BlockSpec((D, R), lambda i: (0, 0)),
            pl.BlockSpec((1, D), lambda i: (0, 0)),
            pl.BlockSpec((D, D), lambda i: (0, 0)),
            pl.BlockSpec((1, D), lambda i: (0, 0)),
        ],
        out_specs=pl.BlockSpec((BE, D), lambda i: (i, 0)),
        out_shape=jax.ShapeDtypeStruct((E, D), jnp.float32),
    )(rbf, Wc1, bc1.reshape(1, D), Wc2, bc2.reshape(1, D))

    src3 = edge_index[0].reshape(NW * CH, 1, K)
    dst3 = edge_index[1].reshape(NW * CH, 1, K)
    zeros = jnp.zeros((N, D), jnp.float32)
    partials = _make_sc(N, D, E)(new_node, h, src3, dst3, zeros)

    BN = 2000
    out = pl.pallas_call(
        _out_body,
        grid=(N // BN,),
        in_specs=[
            pl.BlockSpec((NC, BN, D), lambda i: (0, i, 0)),
            pl.BlockSpec((BN, D), lambda i: (i, 0)),
            pl.BlockSpec((D, D), lambda i: (0, 0)),
            pl.BlockSpec((1, D), lambda i: (0, 0)),
            pl.BlockSpec((D, D), lambda i: (0, 0)),
            pl.BlockSpec((1, D), lambda i: (0, 0)),
        ],
        out_specs=pl.BlockSpec((BN, D), lambda i: (i, 0)),
        out_shape=jax.ShapeDtypeStruct((N, D), jnp.float32),
    )(partials, x, W2, b2.reshape(1, D), W3, b3.reshape(1, D))

    return out


# speedup vs baseline: 3.4345x; 1.2172x over previous
import functools

import jax
import jax.numpy as jnp
from jax import lax
from jax.experimental import pallas as pl
from jax.experimental.pallas import tpu as pltpu
from jax.experimental.pallas import tpu_sc as plsc

NC = 2
NS = 16
NW = NC * NS
K = 40

_LP = (2.55467302e-07, 0.999967081, -0.499285049, 0.327225715,
       -0.223165864, 0.130833428, -0.0524375371, 0.0100092896)


def _ssp(v):
    z = 0.5 * v
    u = jnp.exp(-jnp.abs(z))
    p = jnp.float32(_LP[7])
    for c in _LP[6::-1]:
        p = p * u + jnp.float32(c)
    return 2.0 * (jnp.maximum(z, 0.0) + p)


def _mm_t(a, b):
    return lax.dot_general(a, b, (((1,), (1,)), ((), ())),
                           preferred_element_type=jnp.float32)


def _nn_body(x_ref, w_ref, o_ref):
    o_ref[...] = _mm_t(x_ref[...], w_ref[...])


def _emlp_body(rbf_ref, wc1_ref, bc1_ref, wc2_ref, bc2_ref, h_ref):
    a = _ssp(_mm_t(rbf_ref[...], wc1_ref[...]) + bc1_ref[...])
    h_ref[...] = _mm_t(a, wc2_ref[...]) + bc2_ref[...]


def _out_body(p_ref, x_ref, w2_ref, b2_ref, w3_ref, b3_ref, o_ref):
    cf = p_ref[0] + p_ref[1]
    t = _ssp(_mm_t(cf, w2_ref[...]) + b2_ref[...])
    o_ref[...] = x_ref[...] + _mm_t(t, w3_ref[...]) + b3_ref[...]


@functools.cache
def _make_sc(N, D, E):
    EW = E // NW
    CH = EW // K
    RT = (N // NS) // 8 * 8
    TAIL = N - NS * RT
    mesh = plsc.VectorSubcoreMesh(core_axis_name="c", subcore_axis_name="s")

    @functools.partial(
        pl.kernel,
        out_type=jax.ShapeDtypeStruct((NC, N, D), jnp.float32),
        mesh=mesh,
        scratch_types=[
            pltpu.VMEM((EW,), jnp.int32),
            pltpu.VMEM((2, 1, K), jnp.int32),
            pltpu.VMEM((2, K, D), jnp.float32),
            pltpu.VMEM((2, K, D), jnp.float32),
            pltpu.VMEM_SHARED((N, D), jnp.float32),
            pltpu.SemaphoreType.DMA,
            pltpu.SemaphoreType.DMA,
            pltpu.SemaphoreType.DMA,
            pltpu.SemaphoreType.DMA,
            pltpu.SemaphoreType.DMA,
            pltpu.SemaphoreType.DMA,
        ],
    )
    def sc_fn(nn_hbm, h_hbm, src_hbm, dst_hbm, zero_hbm, out_hbm,
              src_v, dst_v, gath_v, h_v, acc_sh,
              sem_d0, sem_d1, sem_g0, sem_g1, sem_h0, sem_h1):
        cid = lax.axis_index("c")
        sid = lax.axis_index("s")
        wid = cid * NS + sid
        base = wid * EW
        sem_d = (sem_d0, sem_d1)
        sem_g = (sem_g0, sem_g1)
        sem_h = (sem_h0, sem_h1)

        pltpu.sync_copy(zero_hbm.at[pl.ds(sid * RT, RT)],
                        acc_sh.at[pl.ds(sid * RT, RT)])
        @pl.when(sid == NS - 1)
        def _():
            pltpu.sync_copy(zero_hbm.at[pl.ds(NS * RT, TAIL)],
                            acc_sh.at[pl.ds(NS * RT, TAIL)])
        pltpu.sync_copy(src_hbm.at[pl.ds(base, EW)], src_v)
        plsc.subcore_barrier()

        def issue(c, b):
            pltpu.async_copy(dst_hbm.at[wid * CH + c], dst_v.at[b], sem_d[b])
            pltpu.async_copy(h_hbm.at[pl.ds(base + c * K, K)], h_v.at[b],
                             sem_h[b])
            pltpu.async_copy(nn_hbm.at[src_v.at[pl.ds(c * K, K)]],
                             gath_v.at[b], sem_g[b])

        def process(c, b):
            pltpu.make_async_copy(h_hbm.at[pl.ds(base + c * K, K)],
                                  h_v.at[b], sem_h[b]).wait()
            pltpu.make_async_copy(nn_hbm.at[src_v.at[pl.ds(c * K, K)]],
                                  gath_v.at[b], sem_g[b]).wait()

            def row(r, carry2):
                for j in range(D // 16):
                    s = pl.ds(j * 16, 16)
                    gath_v[b, r, s] = gath_v[b, r, s] * h_v[b, r, s]
                return carry2

            lax.fori_loop(0, K, row, 0)

            pltpu.make_async_copy(dst_hbm.at[wid * CH + c],
                                  dst_v.at[b], sem_d[b]).wait()
            pltpu.sync_copy(gath_v.at[b], acc_sh.at[dst_v.at[b, 0]], add=True)

        issue(0, 0)

        def pair(i, carry):
            c0 = 2 * i
            issue(c0 + 1, 1)
            process(c0, 0)

            @pl.when(i < CH // 2 - 1)
            def _():
                issue(c0 + 2, 0)

            process(c0 + 1, 1)
            return carry

        lax.fori_loop(0, CH // 2, pair, 0)
        plsc.subcore_barrier()

        pltpu.sync_copy(acc_sh.at[pl.ds(sid * RT, RT)],
                        out_hbm.at[cid, pl.ds(sid * RT, RT)])
        @pl.when(sid == NS - 1)
        def _():
            pltpu.sync_copy(acc_sh.at[pl.ds(NS * RT, TAIL)],
                            out_hbm.at[cid, pl.ds(NS * RT, TAIL)])

    return sc_fn


def kernel(x, edge_index, rbf, W1, Wc1, bc1, Wc2, bc2, W2, b2, W3, b3):
    N, D = x.shape
    E, R = rbf.shape
    EW = E // NW
    CH = EW // K

    new_node = pl.pallas_call(
        _nn_body,
        out_shape=jax.ShapeDtypeStruct((N, D), jnp.float32),
    )(x, W1)

    BE = 4000
    h = pl.pallas_call(
        _emlp_body,
        grid=(E // BE,),
        in_specs=[
            pl.BlockSpec((BE, R), lambda i: (i, 0)),
            pl.BlockSpec((D, R), lambda i: (0, 0)),
            pl.BlockSpec((1, D), lambda i: (0, 0)),
            pl.BlockSpec((D, D), lambda i: (0, 0)),
            pl.BlockSpec((1, D), lambda i: (0, 0)),
        ],
        out_specs=pl.BlockSpec((BE, D), lambda i: (i, 0)),
        out_shape=jax.ShapeDtypeStruct((E, D), jnp.float32),
    )(rbf, Wc1, bc1.reshape(1, D), Wc2, bc2.reshape(1, D))

    src = edge_index[0]
    dst3 = edge_index[1].reshape(NW * CH, 1, K)
    zeros = jnp.zeros((N, D), jnp.float32)
    partials = _make_sc(N, D, E)(new_node, h, src, dst3, zeros)

    BN = 2000
    out = pl.pallas_call(
        _out_body,
        grid=(N // BN,),
        in_specs=[
            pl.BlockSpec((NC, BN, D), lambda i: (0, i, 0)),
            pl.BlockSpec((BN, D), lambda i: (i, 0)),
            pl.BlockSpec((D, D), lambda i: (0, 0)),
            pl.BlockSpec((1, D), lambda i: (0, 0)),
            pl.BlockSpec((D, D), lambda i: (0, 0)),
            pl.BlockSpec((1, D), lambda i: (0, 0)),
        ],
        out_specs=pl.BlockSpec((BN, D), lambda i: (i, 0)),
        out_shape=jax.ShapeDtypeStruct((N, D), jnp.float32),
    )(partials, x, W2, b2.reshape(1, D), W3, b3.reshape(1, D))

    return out


# speedup vs baseline: 4.4419x; 1.2933x over previous
import functools

import jax
import jax.numpy as jnp
from jax import lax
from jax.experimental import pallas as pl
from jax.experimental.pallas import tpu as pltpu
from jax.experimental.pallas import tpu_sc as plsc

NC = 2
NS = 16
NW = NC * NS
K = 40

_LP = (1.14470976e-05, 0.999166401, -0.48969909, 0.283823183,
       -0.129957198, 0.0298087652)


def _ssp(v):
    z = 0.5 * v
    u = jnp.exp(-jnp.abs(z))
    u2 = u * u
    u4 = u2 * u2
    p = (jnp.float32(_LP[0]) + jnp.float32(_LP[1]) * u
         + (jnp.float32(_LP[2]) + jnp.float32(_LP[3]) * u) * u2
         + (jnp.float32(_LP[4]) + jnp.float32(_LP[5]) * u) * u4)
    return 2.0 * (jnp.maximum(z, 0.0) + p)


def _mm_t(a, b):
    return lax.dot_general(a, b, (((1,), (1,)), ((), ())),
                           preferred_element_type=jnp.float32)


def _nn_body(x_ref, w_ref, o_ref):
    o_ref[...] = _mm_t(x_ref[...], w_ref[...])


def _emlp_body(rbft_ref, wc1_ref, bc1_ref, wc2_ref, bc2_ref, h_ref):
    a = lax.dot_general(rbft_ref[...], wc1_ref[...], (((0,), (1,)), ((), ())),
                        preferred_element_type=jnp.float32)
    a = _ssp(a + bc1_ref[...])
    h_ref[...] = _mm_t(a, wc2_ref[...]) + bc2_ref[...]


def _out_body(p_ref, x_ref, w2_ref, b2_ref, w3_ref, b3_ref, o_ref):
    cf = p_ref[0] + p_ref[1]
    t = _ssp(_mm_t(cf, w2_ref[...]) + b2_ref[...])
    o_ref[...] = x_ref[...] + _mm_t(t, w3_ref[...]) + b3_ref[...]


@functools.cache
def _make_sc(N, D, E):
    EW = E // NW
    CH = EW // K
    RT = (N // NS) // 8 * 8
    TAIL = N - NS * RT
    mesh = plsc.VectorSubcoreMesh(core_axis_name="c", subcore_axis_name="s")

    @functools.partial(
        pl.kernel,
        out_type=jax.ShapeDtypeStruct((NC, N, D), jnp.float32),
        mesh=mesh,
        scratch_types=[
            pltpu.VMEM((EW,), jnp.int32),
            pltpu.VMEM((2, 1, K), jnp.int32),
            pltpu.VMEM((2, K, D), jnp.float32),
            pltpu.VMEM((2, K, D), jnp.float32),
            pltpu.VMEM_SHARED((N, D), jnp.float32),
            pltpu.SemaphoreType.DMA,
            pltpu.SemaphoreType.DMA,
            pltpu.SemaphoreType.DMA,
            pltpu.SemaphoreType.DMA,
            pltpu.SemaphoreType.DMA,
            pltpu.SemaphoreType.DMA,
        ],
    )
    def sc_fn(nn_hbm, h_hbm, src_hbm, dst_hbm, zero_hbm, out_hbm,
              src_v, dst_v, gath_v, h_v, acc_sh,
              sem_d0, sem_d1, sem_g0, sem_g1, sem_h0, sem_h1):
        cid = lax.axis_index("c")
        sid = lax.axis_index("s")
        wid = cid * NS + sid
        base = wid * EW
        sem_d = (sem_d0, sem_d1)
        sem_g = (sem_g0, sem_g1)
        sem_h = (sem_h0, sem_h1)

        pltpu.sync_copy(zero_hbm.at[pl.ds(sid * RT, RT)],
                        acc_sh.at[pl.ds(sid * RT, RT)])
        @pl.when(sid == NS - 1)
        def _():
            pltpu.sync_copy(zero_hbm.at[pl.ds(NS * RT, TAIL)],
                            acc_sh.at[pl.ds(NS * RT, TAIL)])
        pltpu.sync_copy(src_hbm.at[pl.ds(base, EW)], src_v)
        plsc.subcore_barrier()

        def issue(c, b):
            pltpu.async_copy(dst_hbm.at[wid * CH + c], dst_v.at[b], sem_d[b])
            pltpu.async_copy(h_hbm.at[pl.ds(base + c * K, K)], h_v.at[b],
                             sem_h[b])
            pltpu.async_copy(nn_hbm.at[src_v.at[pl.ds(c * K, K)]],
                             gath_v.at[b], sem_g[b])

        def process(c, b):
            pltpu.make_async_copy(h_hbm.at[pl.ds(base + c * K, K)],
                                  h_v.at[b], sem_h[b]).wait()
            pltpu.make_async_copy(nn_hbm.at[src_v.at[pl.ds(c * K, K)]],
                                  gath_v.at[b], sem_g[b]).wait()

            def row(r, carry2):
                for j in range(D // 16):
                    s = pl.ds(j * 16, 16)
                    gath_v[b, r, s] = gath_v[b, r, s] * h_v[b, r, s]
                return carry2

            lax.fori_loop(0, K, row, 0)

            pltpu.make_async_copy(dst_hbm.at[wid * CH + c],
                                  dst_v.at[b], sem_d[b]).wait()
            pltpu.sync_copy(gath_v.at[b], acc_sh.at[dst_v.at[b, 0]], add=True)

        issue(0, 0)

        def pair(i, carry):
            c0 = 2 * i
            issue(c0 + 1, 1)
            process(c0, 0)

            @pl.when(i < CH // 2 - 1)
            def _():
                issue(c0 + 2, 0)

            process(c0 + 1, 1)
            return carry

        lax.fori_loop(0, CH // 2, pair, 0)
        plsc.subcore_barrier()

        pltpu.sync_copy(acc_sh.at[pl.ds(sid * RT, RT)],
                        out_hbm.at[cid, pl.ds(sid * RT, RT)])
        @pl.when(sid == NS - 1)
        def _():
            pltpu.sync_copy(acc_sh.at[pl.ds(NS * RT, TAIL)],
                            out_hbm.at[cid, pl.ds(NS * RT, TAIL)])

    return sc_fn


def kernel(x, edge_index, rbf, W1, Wc1, bc1, Wc2, bc2, W2, b2, W3, b3):
    N, D = x.shape
    E, R = rbf.shape
    EW = E // NW
    CH = EW // K

    new_node = pl.pallas_call(
        _nn_body,
        out_shape=jax.ShapeDtypeStruct((N, D), jnp.float32),
    )(x, W1)

    BE = 4096
    h = pl.pallas_call(
        _emlp_body,
        grid=((E + BE - 1) // BE,),
        in_specs=[
            pl.BlockSpec((R, BE), lambda i: (0, i)),
            pl.BlockSpec((D, R), lambda i: (0, 0)),
            pl.BlockSpec((1, D), lambda i: (0, 0)),
            pl.BlockSpec((D, D), lambda i: (0, 0)),
            pl.BlockSpec((1, D), lambda i: (0, 0)),
        ],
        out_specs=pl.BlockSpec((BE, D), lambda i: (i, 0)),
        out_shape=jax.ShapeDtypeStruct((E, D), jnp.float32),
    )(rbf.T, Wc1, bc1.reshape(1, D), Wc2, bc2.reshape(1, D))

    src = edge_index[0]
    dst3 = edge_index[1].reshape(NW * CH, 1, K)
    zeros = jnp.zeros((N, D), jnp.float32)
    partials = _make_sc(N, D, E)(new_node, h, src, dst3, zeros)

    BN = 2000
    out = pl.pallas_call(
        _out_body,
        grid=(N // BN,),
        in_specs=[
            pl.BlockSpec((NC, BN, D), lambda i: (0, i, 0)),
            pl.BlockSpec((BN, D), lambda i: (i, 0)),
            pl.BlockSpec((D, D), lambda i: (0, 0)),
            pl.BlockSpec((1, D), lambda i: (0, 0)),
            pl.BlockSpec((D, D), lambda i: (0, 0)),
            pl.BlockSpec((1, D), lambda i: (0, 0)),
        ],
        out_specs=pl.BlockSpec((BN, D), lambda i: (i, 0)),
        out_shape=jax.ShapeDtypeStruct((N, D), jnp.float32),
    )(partials, x, W2, b2.reshape(1, D), W3, b3.reshape(1, D))

    return out
